# baseline (speedup 1.0000x reference)
import functools

import jax
import jax.numpy as jnp
from jax.experimental import pallas as pl
from jax.experimental.pallas import tpu as pltpu

DIM = 1024
HIDDEN = 2816
E = 8
K = 2

TM = 128
TH = 256
NH = HIDDEN // TH


def _ffn_kernel(te_ref, ntot_ref, xs_ref, ws_ref, w1_ref, w3_ref, w2_ref,
                ys_ref):
    nh = pl.program_id(0)
    mt = pl.program_id(1)
    rows = pl.ds(mt * TM, TM)

    @pl.when(nh == 0)
    def _init():
        ys_ref[rows, :] = jnp.zeros((TM, DIM), jnp.float32)

    @pl.when(mt < ntot_ref[0])
    def _compute():
        xb = xs_ref[rows, :].astype(jnp.bfloat16)
        w1 = w1_ref[0]
        w3 = w3_ref[0]
        dn = (((1,), (1,)), ((), ()))
        h1 = jax.lax.dot_general(xb, w1, dn,
                                 preferred_element_type=jnp.float32)
        h3 = jax.lax.dot_general(xb, w3, dn,
                                 preferred_element_type=jnp.float32)
        h = (h1 * jax.nn.sigmoid(h1)) * h3
        h = h * ws_ref[rows, :]
        out = jax.lax.dot_general(h.astype(jnp.bfloat16), w2_ref[0], dn,
                                  preferred_element_type=jnp.float32)
        ys_ref[rows, :] += out


def _ffn_call(te, ntot, xs, wsort, W1b, W2b, W3b, mp, nt, interpret=False):
    grid_spec = pltpu.PrefetchScalarGridSpec(
        num_scalar_prefetch=2,
        grid=(NH, nt),
        in_specs=[
            pl.BlockSpec((mp, DIM), lambda nh, mt, te, ntot: (0, 0)),
            pl.BlockSpec((mp, 1), lambda nh, mt, te, ntot: (0, 0)),
            pl.BlockSpec((1, TH, DIM), lambda nh, mt, te, ntot:
                         (te[mt], nh, 0)),
            pl.BlockSpec((1, TH, DIM), lambda nh, mt, te, ntot:
                         (te[mt], nh, 0)),
            pl.BlockSpec((1, DIM, TH), lambda nh, mt, te, ntot:
                         (te[mt], 0, nh)),
        ],
        out_specs=pl.BlockSpec((mp, DIM), lambda nh, mt, te, ntot: (0, 0)),
    )
    return pl.pallas_call(
        _ffn_kernel,
        grid_spec=grid_spec,
        out_shape=jax.ShapeDtypeStruct((mp, DIM), jnp.float32),
        compiler_params=pltpu.CompilerParams(
            dimension_semantics=("arbitrary", "arbitrary")),
        interpret=interpret,
    )(te, ntot, xs, wsort, W1b, W3b, W2b)


def _route(xt, gate_w, n):
    logits = xt @ gate_w.T
    scores = jax.nn.softmax(logits, axis=-1)
    topw, topi = jax.lax.top_k(scores, K)
    topw = topw / (jnp.sum(topw, axis=-1, keepdims=True) + 1e-20)
    e_flat = topi.T.reshape(-1)
    w_flat = topw.T.reshape(-1)
    onehot = (e_flat[:, None] == jnp.arange(E)[None, :]).astype(jnp.int32)
    ranks = jnp.cumsum(onehot, axis=0) - onehot
    rank = jnp.sum(ranks * onehot, axis=-1)
    counts = jnp.sum(onehot, axis=0)
    ntiles = (counts + TM - 1) // TM
    tile_start = jnp.concatenate(
        [jnp.zeros((1,), jnp.int32), jnp.cumsum(ntiles)[:-1]])
    pos = tile_start[e_flat] * TM + rank
    ntot = jnp.sum(ntiles)
    nt = n * K // TM + (E - 1)
    j = jnp.arange(nt)
    in_e = (j[:, None] >= tile_start[None, :]) & (
        j[:, None] < (tile_start + ntiles)[None, :])
    te_raw = jnp.sum(jnp.arange(E)[None, :] * in_e, axis=-1).astype(jnp.int32)
    te_last = jnp.sum(jnp.where(j == ntot - 1, te_raw, 0))
    te = jnp.where(j < ntot, te_raw, te_last).astype(jnp.int32)
    return pos, w_flat, te, ntot.astype(jnp.int32).reshape(1)


def kernel(x, W1, W2, W3, gate_w):
    orig_shape = x.shape
    xt = x.reshape(-1, DIM)
    n = xt.shape[0]
    nt = n * K // TM + (E - 1)
    mp = nt * TM

    pos, w_flat, te, ntot = _route(xt, gate_w, n)

    xs = jnp.zeros((mp, DIM), jnp.float32).at[pos, :].set(
        jnp.concatenate([xt, xt], axis=0))
    wsort = jnp.zeros((mp, 1), jnp.float32).at[pos, 0].set(w_flat)

    W1b = W1.astype(jnp.bfloat16)
    W2b = W2.astype(jnp.bfloat16)
    W3b = W3.astype(jnp.bfloat16)

    ys = _ffn_call(te, ntot, xs, wsort, W1b, W2b, W3b, mp, nt)

    y = ys[pos[:n]] + ys[pos[n:]]
    return y.reshape(orig_shape)


# speedup vs baseline: 1.0472x; 1.0472x over previous
import functools

import jax
import jax.numpy as jnp
from jax import lax
from jax.experimental import pallas as pl
from jax.experimental.pallas import tpu as pltpu
from jax.experimental.pallas import tpu_sc as plsc

DIM = 1024
HIDDEN = 2816
E = 8
K = 2

TM = 128
TH = 256
NH = HIDDEN // TH

NC = 2
NS = 16
NW = NC * NS


def _ffn_kernel(te_ref, ntot_ref, xs_ref, ws_ref, w1_ref, w3_ref, w2_ref,
                ys_ref):
    nh = pl.program_id(0)
    mt = pl.program_id(1)
    rows = pl.ds(mt * TM, TM)

    @pl.when(nh == 0)
    def _init():
        ys_ref[rows, :] = jnp.zeros((TM, DIM), jnp.float32)

    @pl.when(mt < ntot_ref[0])
    def _compute():
        xb = xs_ref[rows, :].astype(jnp.bfloat16)
        w1 = w1_ref[0]
        w3 = w3_ref[0]
        dn = (((1,), (1,)), ((), ()))
        h1 = jax.lax.dot_general(xb, w1, dn,
                                 preferred_element_type=jnp.float32)
        h3 = jax.lax.dot_general(xb, w3, dn,
                                 preferred_element_type=jnp.float32)
        h = (h1 * jax.nn.sigmoid(h1)) * h3
        h = h * ws_ref[rows, :]
        out = jax.lax.dot_general(h.astype(jnp.bfloat16), w2_ref[0], dn,
                                  preferred_element_type=jnp.float32)
        ys_ref[rows, :] += out


def _ffn_call(te, ntot, xs, wsort, W1b, W2b, W3b, mp, nt, interpret=False):
    grid_spec = pltpu.PrefetchScalarGridSpec(
        num_scalar_prefetch=2,
        grid=(NH, nt),
        in_specs=[
            pl.BlockSpec((mp, DIM), lambda nh, mt, te, ntot: (0, 0)),
            pl.BlockSpec((mp, 1), lambda nh, mt, te, ntot: (0, 0)),
            pl.BlockSpec((1, TH, DIM), lambda nh, mt, te, ntot:
                         (te[mt], nh, 0)),
            pl.BlockSpec((1, TH, DIM), lambda nh, mt, te, ntot:
                         (te[mt], nh, 0)),
            pl.BlockSpec((1, DIM, TH), lambda nh, mt, te, ntot:
                         (te[mt], 0, nh)),
        ],
        out_specs=pl.BlockSpec((mp, DIM), lambda nh, mt, te, ntot: (0, 0)),
    )
    return pl.pallas_call(
        _ffn_kernel,
        grid_spec=grid_spec,
        out_shape=jax.ShapeDtypeStruct((mp, DIM), jnp.float32),
        compiler_params=pltpu.CompilerParams(
            dimension_semantics=("arbitrary", "arbitrary")),
        interpret=interpret,
    )(te, ntot, xs, wsort, W1b, W3b, W2b)


def _sc_dispatch(xt, pos2, mp, n):
    tpw = n // NW
    mesh = plsc.VectorSubcoreMesh(core_axis_name="c", subcore_axis_name="s")

    @functools.partial(
        pl.kernel, mesh=mesh,
        out_type=jax.ShapeDtypeStruct((mp, DIM), jnp.float32),
        scratch_types=[
            pltpu.VMEM((tpw,), jnp.int32),
            pltpu.VMEM((tpw,), jnp.int32),
            pltpu.VMEM((tpw, DIM), jnp.float32),
            pltpu.SemaphoreType.DMA,
        ],
    )
    def dispatch(x_hbm, pos_hbm, xs_hbm, idx0_v, idx1_v, rows_v, sem):
        wid = lax.axis_index("s") * NC + lax.axis_index("c")
        base = wid * tpw
        pltpu.sync_copy(pos_hbm.at[0, pl.ds(base, tpw)], idx0_v)
        pltpu.sync_copy(pos_hbm.at[1, pl.ds(base, tpw)], idx1_v)
        pltpu.sync_copy(x_hbm.at[pl.ds(base, tpw)], rows_v)
        pltpu.async_copy(rows_v, xs_hbm.at[idx0_v], sem).wait()
        pltpu.async_copy(rows_v, xs_hbm.at[idx1_v], sem).wait()

    return dispatch(xt, pos2)


def _sc_combine(ys, pos2, n):
    tpw = n // NW
    cch = tpw // 2
    mesh = plsc.VectorSubcoreMesh(core_axis_name="c", subcore_axis_name="s")

    @functools.partial(
        pl.kernel, mesh=mesh,
        out_type=jax.ShapeDtypeStruct((n, DIM), jnp.float32),
        scratch_types=[
            pltpu.VMEM((cch,), jnp.int32),
            pltpu.VMEM((cch,), jnp.int32),
            pltpu.VMEM((cch, DIM), jnp.float32),
            pltpu.VMEM((cch, DIM), jnp.float32),
            pltpu.SemaphoreType.DMA,
        ],
    )
    def combine(ys_hbm, pos_hbm, y_hbm, idx0_v, idx1_v, buf0, buf1, sem):
        wid = lax.axis_index("s") * NC + lax.axis_index("c")
        for c in range(2):
            base = wid * tpw + c * cch
            pltpu.sync_copy(pos_hbm.at[0, pl.ds(base, cch)], idx0_v)
            pltpu.sync_copy(pos_hbm.at[1, pl.ds(base, cch)], idx1_v)
            pltpu.async_copy(ys_hbm.at[idx0_v], buf0, sem).wait()
            pltpu.async_copy(ys_hbm.at[idx1_v], buf1, sem).wait()

            def row(i, _):
                def col(j, _):
                    sl = pl.ds(j * 16, 16)
                    buf0[i, sl] = buf0[i, sl] + buf1[i, sl]
                    return 0
                lax.fori_loop(0, DIM // 16, col, 0, unroll=8)
                return 0
            lax.fori_loop(0, cch, row, 0)
            pltpu.sync_copy(buf0, y_hbm.at[pl.ds(base, cch)])

    return combine(ys, pos2)


def _route(xt, gate_w, n):
    logits = xt @ gate_w.T
    scores = jax.nn.softmax(logits, axis=-1)
    topw, topi = jax.lax.top_k(scores, K)
    topw = topw / (jnp.sum(topw, axis=-1, keepdims=True) + 1e-20)
    e_flat = topi.T.reshape(-1)
    w_flat = topw.T.reshape(-1)
    onehot = (e_flat[:, None] == jnp.arange(E)[None, :]).astype(jnp.int32)
    ranks = jnp.cumsum(onehot, axis=0) - onehot
    rank = jnp.sum(ranks * onehot, axis=-1)
    counts = jnp.sum(onehot, axis=0)
    ntiles = (counts + TM - 1) // TM
    tile_start = jnp.concatenate(
        [jnp.zeros((1,), jnp.int32), jnp.cumsum(ntiles)[:-1]])
    pos = tile_start[e_flat] * TM + rank
    ntot = jnp.sum(ntiles)
    nt = n * K // TM + (E - 1)
    j = jnp.arange(nt)
    in_e = (j[:, None] >= tile_start[None, :]) & (
        j[:, None] < (tile_start + ntiles)[None, :])
    te_raw = jnp.sum(jnp.arange(E)[None, :] * in_e, axis=-1).astype(jnp.int32)
    te_last = jnp.sum(jnp.where(j == ntot - 1, te_raw, 0))
    te = jnp.where(j < ntot, te_raw, te_last).astype(jnp.int32)
    return pos, w_flat, te, ntot.astype(jnp.int32).reshape(1)


def kernel(x, W1, W2, W3, gate_w):
    orig_shape = x.shape
    xt = x.reshape(-1, DIM)
    n = xt.shape[0]
    nt = n * K // TM + (E - 1)
    mp = nt * TM

    pos, w_flat, te, ntot = _route(xt, gate_w, n)
    pos2 = pos.reshape(K, n)

    xs = _sc_dispatch(xt, pos2, mp, n)
    wsort = jnp.zeros((mp, 1), jnp.float32).at[pos, 0].set(w_flat)

    W1b = W1.astype(jnp.bfloat16)
    W2b = W2.astype(jnp.bfloat16)
    W3b = W3.astype(jnp.bfloat16)

    ys = _ffn_call(te, ntot, xs, wsort, W1b, W2b, W3b, mp, nt)

    y = _sc_combine(ys, pos2, n)
    return y.reshape(orig_shape)


# speedup vs baseline: 1.4772x; 1.4106x over previous
import functools

import jax
import jax.numpy as jnp
from jax import lax
from jax.experimental import pallas as pl
from jax.experimental.pallas import tpu as pltpu
from jax.experimental.pallas import tpu_sc as plsc

DIM = 1024
HIDDEN = 2816
E = 8
K = 2

TM = 128
TH = 256
NH = HIDDEN // TH

NC = 2
NS = 16
NW = NC * NS


def _ffn_kernel(te_ref, ntot_ref, xs_ref, ws_ref, w1_ref, w3_ref, w2_ref,
                ys_ref):
    mt = pl.program_id(0)

    @pl.when(mt >= ntot_ref[0])
    def _dead():
        ys_ref[...] = jnp.zeros((TM, DIM), jnp.float32)

    @pl.when(mt < ntot_ref[0])
    def _compute():
        xb = xs_ref[...].astype(jnp.bfloat16)
        dn = (((1,), (1,)), ((), ()))
        h1 = jax.lax.dot_general(xb, w1_ref[0], dn,
                                 preferred_element_type=jnp.float32)
        h3 = jax.lax.dot_general(xb, w3_ref[0], dn,
                                 preferred_element_type=jnp.float32)
        h = (h1 * jax.nn.sigmoid(h1)) * h3
        h = h * ws_ref[...]
        ys_ref[...] = jax.lax.dot_general(
            h.astype(jnp.bfloat16), w2_ref[0], dn,
            preferred_element_type=jnp.float32)


def _ffn_call(te, ntot, xs, wsort, W1b, W2b, W3b, mp, nt, interpret=False):
    grid_spec = pltpu.PrefetchScalarGridSpec(
        num_scalar_prefetch=2,
        grid=(nt,),
        in_specs=[
            pl.BlockSpec((TM, DIM), lambda mt, te, ntot: (mt, 0)),
            pl.BlockSpec((TM, 1), lambda mt, te, ntot: (mt, 0)),
            pl.BlockSpec((1, HIDDEN, DIM), lambda mt, te, ntot:
                         (te[mt], 0, 0)),
            pl.BlockSpec((1, HIDDEN, DIM), lambda mt, te, ntot:
                         (te[mt], 0, 0)),
            pl.BlockSpec((1, DIM, HIDDEN), lambda mt, te, ntot:
                         (te[mt], 0, 0)),
        ],
        out_specs=pl.BlockSpec((TM, DIM), lambda mt, te, ntot: (mt, 0)),
    )
    return pl.pallas_call(
        _ffn_kernel,
        grid_spec=grid_spec,
        out_shape=jax.ShapeDtypeStruct((mp, DIM), jnp.float32),
        compiler_params=pltpu.CompilerParams(
            dimension_semantics=("arbitrary",)),
        interpret=interpret,
    )(te, ntot, xs, wsort, W1b, W3b, W2b)


def _sc_dispatch(xt, pos2, mp, n):
    tpw = n // NW
    mesh = plsc.VectorSubcoreMesh(core_axis_name="c", subcore_axis_name="s")

    @functools.partial(
        pl.kernel, mesh=mesh,
        out_type=jax.ShapeDtypeStruct((mp, DIM), jnp.float32),
        scratch_types=[
            pltpu.VMEM((tpw,), jnp.int32),
            pltpu.VMEM((tpw,), jnp.int32),
            pltpu.VMEM((tpw, DIM), jnp.float32),
            pltpu.SemaphoreType.DMA,
        ],
    )
    def dispatch(x_hbm, pos_hbm, xs_hbm, idx0_v, idx1_v, rows_v, sem):
        wid = lax.axis_index("s") * NC + lax.axis_index("c")
        base = wid * tpw
        pltpu.sync_copy(pos_hbm.at[0, pl.ds(base, tpw)], idx0_v)
        pltpu.sync_copy(pos_hbm.at[1, pl.ds(base, tpw)], idx1_v)
        pltpu.sync_copy(x_hbm.at[pl.ds(base, tpw)], rows_v)
        pltpu.async_copy(rows_v, xs_hbm.at[idx0_v], sem).wait()
        pltpu.async_copy(rows_v, xs_hbm.at[idx1_v], sem).wait()

    return dispatch(xt, pos2)


def _sc_combine(ys, pos2, n):
    tpw = n // NW
    cch = tpw // 2
    mesh = plsc.VectorSubcoreMesh(core_axis_name="c", subcore_axis_name="s")

    @functools.partial(
        pl.kernel, mesh=mesh,
        out_type=jax.ShapeDtypeStruct((n, DIM), jnp.float32),
        scratch_types=[
            pltpu.VMEM((cch,), jnp.int32),
            pltpu.VMEM((cch,), jnp.int32),
            pltpu.VMEM((cch, DIM), jnp.float32),
            pltpu.VMEM((cch, DIM), jnp.float32),
            pltpu.SemaphoreType.DMA,
        ],
    )
    def combine(ys_hbm, pos_hbm, y_hbm, idx0_v, idx1_v, buf0, buf1, sem):
        wid = lax.axis_index("s") * NC + lax.axis_index("c")
        for c in range(2):
            base = wid * tpw + c * cch
            pltpu.sync_copy(pos_hbm.at[0, pl.ds(base, cch)], idx0_v)
            pltpu.sync_copy(pos_hbm.at[1, pl.ds(base, cch)], idx1_v)
            pltpu.async_copy(ys_hbm.at[idx0_v], buf0, sem).wait()
            pltpu.async_copy(ys_hbm.at[idx1_v], buf1, sem).wait()

            def row(i, _):
                def col(j, _):
                    sl = pl.ds(j * 16, 16)
                    buf0[i, sl] = buf0[i, sl] + buf1[i, sl]
                    return 0
                lax.fori_loop(0, DIM // 16, col, 0, unroll=8)
                return 0
            lax.fori_loop(0, cch, row, 0)
            pltpu.sync_copy(buf0, y_hbm.at[pl.ds(base, cch)])

    return combine(ys, pos2)


def _route(xt, gate_w, n):
    logits = xt @ gate_w.T
    scores = jax.nn.softmax(logits, axis=-1)
    topw, topi = jax.lax.top_k(scores, K)
    topw = topw / (jnp.sum(topw, axis=-1, keepdims=True) + 1e-20)
    e_flat = topi.T.reshape(-1)
    w_flat = topw.T.reshape(-1)
    onehot = (e_flat[:, None] == jnp.arange(E)[None, :]).astype(jnp.int32)
    ranks = jnp.cumsum(onehot, axis=0) - onehot
    rank = jnp.sum(ranks * onehot, axis=-1)
    counts = jnp.sum(onehot, axis=0)
    ntiles = (counts + TM - 1) // TM
    tile_start = jnp.concatenate(
        [jnp.zeros((1,), jnp.int32), jnp.cumsum(ntiles)[:-1]])
    pos = tile_start[e_flat] * TM + rank
    ntot = jnp.sum(ntiles)
    nt = n * K // TM + (E - 1)
    j = jnp.arange(nt)
    in_e = (j[:, None] >= tile_start[None, :]) & (
        j[:, None] < (tile_start + ntiles)[None, :])
    te_raw = jnp.sum(jnp.arange(E)[None, :] * in_e, axis=-1).astype(jnp.int32)
    te_last = jnp.sum(jnp.where(j == ntot - 1, te_raw, 0))
    te = jnp.where(j < ntot, te_raw, te_last).astype(jnp.int32)
    return pos, w_flat, te, ntot.astype(jnp.int32).reshape(1)


def kernel(x, W1, W2, W3, gate_w):
    orig_shape = x.shape
    xt = x.reshape(-1, DIM)
    n = xt.shape[0]
    nt = n * K // TM + (E - 1)
    mp = nt * TM

    pos, w_flat, te, ntot = _route(xt, gate_w, n)
    pos2 = pos.reshape(K, n)

    xs = _sc_dispatch(xt, pos2, mp, n)
    wsort = jnp.zeros((mp, 1), jnp.float32).at[pos, 0].set(w_flat)

    W1b = W1.astype(jnp.bfloat16)
    W2b = W2.astype(jnp.bfloat16)
    W3b = W3.astype(jnp.bfloat16)

    ys = _ffn_call(te, ntot, xs, wsort, W1b, W2b, W3b, mp, nt)

    y = _sc_combine(ys, pos2, n)
    return y.reshape(orig_shape)


# speedup vs baseline: 1.5311x; 1.0365x over previous
import functools

import jax
import jax.numpy as jnp
from jax import lax
from jax.experimental import pallas as pl
from jax.experimental.pallas import tpu as pltpu
from jax.experimental.pallas import tpu_sc as plsc

DIM = 1024
HIDDEN = 2816
E = 8
K = 2

TM = 128
TH = 256
NH = HIDDEN // TH

NC = 2
NS = 16
NW = NC * NS


def _ffn_kernel(te_ref, ntot_ref, xs_ref, ws_ref, w1_ref, w3_ref, w2_ref,
                ys_ref):
    mt = pl.program_id(0)

    @pl.when(mt >= ntot_ref[0])
    def _dead():
        ys_ref[...] = jnp.zeros((TM, DIM), jnp.float32)

    @pl.when(mt < ntot_ref[0])
    def _compute():
        xb = xs_ref[...].astype(jnp.bfloat16)
        dn = (((1,), (0,)), ((), ()))
        h1 = jax.lax.dot_general(xb, w1_ref[0], dn,
                                 preferred_element_type=jnp.float32)
        h3 = jax.lax.dot_general(xb, w3_ref[0], dn,
                                 preferred_element_type=jnp.float32)
        h = (h1 * jax.nn.sigmoid(h1)) * h3
        h = h * ws_ref[...]
        ys_ref[...] = jax.lax.dot_general(
            h.astype(jnp.bfloat16), w2_ref[0], dn,
            preferred_element_type=jnp.float32)


def _ffn_call(te, ntot, xs, wsort, W1b, W2b, W3b, mp, nt, interpret=False):
    grid_spec = pltpu.PrefetchScalarGridSpec(
        num_scalar_prefetch=2,
        grid=(nt,),
        in_specs=[
            pl.BlockSpec((TM, DIM), lambda mt, te, ntot: (mt, 0)),
            pl.BlockSpec((TM, 1), lambda mt, te, ntot: (mt, 0)),
            pl.BlockSpec((1, DIM, HIDDEN), lambda mt, te, ntot:
                         (te[mt], 0, 0)),
            pl.BlockSpec((1, DIM, HIDDEN), lambda mt, te, ntot:
                         (te[mt], 0, 0)),
            pl.BlockSpec((1, HIDDEN, DIM), lambda mt, te, ntot:
                         (te[mt], 0, 0)),
        ],
        out_specs=pl.BlockSpec((TM, DIM), lambda mt, te, ntot: (mt, 0)),
    )
    return pl.pallas_call(
        _ffn_kernel,
        grid_spec=grid_spec,
        out_shape=jax.ShapeDtypeStruct((mp, DIM), jnp.float32),
        compiler_params=pltpu.CompilerParams(
            dimension_semantics=("arbitrary",)),
        interpret=interpret,
    )(te, ntot, xs, wsort, W1b, W3b, W2b)


def _sc_dispatch(xt, pos2, mp, n):
    tpw = n // NW
    mesh = plsc.VectorSubcoreMesh(core_axis_name="c", subcore_axis_name="s")

    @functools.partial(
        pl.kernel, mesh=mesh,
        out_type=jax.ShapeDtypeStruct((mp, DIM), jnp.float32),
        scratch_types=[
            pltpu.VMEM((tpw,), jnp.int32),
            pltpu.VMEM((tpw,), jnp.int32),
            pltpu.VMEM((tpw, DIM), jnp.float32),
            pltpu.SemaphoreType.DMA,
        ],
    )
    def dispatch(x_hbm, pos_hbm, xs_hbm, idx0_v, idx1_v, rows_v, sem):
        wid = lax.axis_index("s") * NC + lax.axis_index("c")
        base = wid * tpw
        pltpu.sync_copy(pos_hbm.at[0, pl.ds(base, tpw)], idx0_v)
        pltpu.sync_copy(pos_hbm.at[1, pl.ds(base, tpw)], idx1_v)
        pltpu.sync_copy(x_hbm.at[pl.ds(base, tpw)], rows_v)
        pltpu.async_copy(rows_v, xs_hbm.at[idx0_v], sem).wait()
        pltpu.async_copy(rows_v, xs_hbm.at[idx1_v], sem).wait()

    return dispatch(xt, pos2)


def _sc_combine(ys, pos2, n):
    tpw = n // NW
    cch = tpw // 2
    mesh = plsc.VectorSubcoreMesh(core_axis_name="c", subcore_axis_name="s")

    @functools.partial(
        pl.kernel, mesh=mesh,
        out_type=jax.ShapeDtypeStruct((n, DIM), jnp.float32),
        scratch_types=[
            pltpu.VMEM((cch,), jnp.int32),
            pltpu.VMEM((cch,), jnp.int32),
            pltpu.VMEM((cch, DIM), jnp.float32),
            pltpu.VMEM((cch, DIM), jnp.float32),
            pltpu.SemaphoreType.DMA,
        ],
    )
    def combine(ys_hbm, pos_hbm, y_hbm, idx0_v, idx1_v, buf0, buf1, sem):
        wid = lax.axis_index("s") * NC + lax.axis_index("c")
        for c in range(2):
            base = wid * tpw + c * cch
            pltpu.sync_copy(pos_hbm.at[0, pl.ds(base, cch)], idx0_v)
            pltpu.sync_copy(pos_hbm.at[1, pl.ds(base, cch)], idx1_v)
            pltpu.async_copy(ys_hbm.at[idx0_v], buf0, sem).wait()
            pltpu.async_copy(ys_hbm.at[idx1_v], buf1, sem).wait()

            def row(i, _):
                def col(j, _):
                    sl = pl.ds(j * 16, 16)
                    buf0[i, sl] = buf0[i, sl] + buf1[i, sl]
                    return 0
                lax.fori_loop(0, DIM // 16, col, 0, unroll=8)
                return 0
            lax.fori_loop(0, cch, row, 0)
            pltpu.sync_copy(buf0, y_hbm.at[pl.ds(base, cch)])

    return combine(ys, pos2)


def _route(xt, gate_w, n):
    logits = xt @ gate_w.T
    scores = jax.nn.softmax(logits, axis=-1)
    topw, topi = jax.lax.top_k(scores, K)
    topw = topw / (jnp.sum(topw, axis=-1, keepdims=True) + 1e-20)
    e_flat = topi.T.reshape(-1)
    w_flat = topw.T.reshape(-1)
    onehot = (e_flat[:, None] == jnp.arange(E)[None, :]).astype(jnp.int32)
    ranks = jnp.cumsum(onehot, axis=0) - onehot
    rank = jnp.sum(ranks * onehot, axis=-1)
    counts = jnp.sum(onehot, axis=0)
    ntiles = (counts + TM - 1) // TM
    tile_start = jnp.concatenate(
        [jnp.zeros((1,), jnp.int32), jnp.cumsum(ntiles)[:-1]])
    pos = tile_start[e_flat] * TM + rank
    ntot = jnp.sum(ntiles)
    nt = n * K // TM + (E - 1)
    j = jnp.arange(nt)
    in_e = (j[:, None] >= tile_start[None, :]) & (
        j[:, None] < (tile_start + ntiles)[None, :])
    te_raw = jnp.sum(jnp.arange(E)[None, :] * in_e, axis=-1).astype(jnp.int32)
    te_last = jnp.sum(jnp.where(j == ntot - 1, te_raw, 0))
    te = jnp.where(j < ntot, te_raw, te_last).astype(jnp.int32)
    return pos, w_flat, te, ntot.astype(jnp.int32).reshape(1)


def kernel(x, W1, W2, W3, gate_w):
    orig_shape = x.shape
    xt = x.reshape(-1, DIM)
    n = xt.shape[0]
    nt = n * K // TM + (E - 1)
    mp = nt * TM

    pos, w_flat, te, ntot = _route(xt, gate_w, n)
    pos2 = pos.reshape(K, n)

    xs = _sc_dispatch(xt, pos2, mp, n)
    wsort = jnp.zeros((mp, 1), jnp.float32).at[pos, 0].set(w_flat)

    W1b = W1.swapaxes(1, 2).astype(jnp.bfloat16)
    W2b = W2.swapaxes(1, 2).astype(jnp.bfloat16)
    W3b = W3.swapaxes(1, 2).astype(jnp.bfloat16)

    ys = _ffn_call(te, ntot, xs, wsort, W1b, W2b, W3b, mp, nt)

    y = _sc_combine(ys, pos2, n)
    return y.reshape(orig_shape)


# speedup vs baseline: 1.9802x; 1.2933x over previous
import functools

import jax
import jax.numpy as jnp
from jax import lax
from jax.experimental import pallas as pl
from jax.experimental.pallas import tpu as pltpu
from jax.experimental.pallas import tpu_sc as plsc

DIM = 1024
HIDDEN = 2816
E = 8
K = 2

TM = 128
TH = 1408
NH = HIDDEN // TH

NC = 2
NS = 16
NW = NC * NS


def _ffn_kernel(te_ref, ntot_ref, xs_ref, ws_ref, w1_ref, w3_ref, w2_ref,
                ys_ref, w1s, w3s, w2s):
    nh = pl.program_id(0)
    mt = pl.program_id(1)

    @pl.when((mt >= ntot_ref[0]) & (nh == 0))
    def _dead():
        ys_ref[...] = jnp.zeros((TM, DIM), jnp.float32)

    prev = te_ref[jnp.maximum(mt - 1, 0)]
    changed = (mt == 0) | (te_ref[mt] != prev)

    @pl.when(changed & (mt < ntot_ref[0]))
    def _cast():
        w1s[...] = w1_ref[0].astype(jnp.bfloat16).T
        w3s[...] = w3_ref[0].astype(jnp.bfloat16).T
        w2s[...] = w2_ref[0].astype(jnp.bfloat16).T

    @pl.when(mt < ntot_ref[0])
    def _compute():
        xb = xs_ref[...].astype(jnp.bfloat16)
        dn = (((1,), (0,)), ((), ()))
        h1 = jax.lax.dot_general(xb, w1s[...], dn,
                                 preferred_element_type=jnp.float32)
        h3 = jax.lax.dot_general(xb, w3s[...], dn,
                                 preferred_element_type=jnp.float32)
        h = (h1 * jax.nn.sigmoid(h1)) * h3
        h = h * ws_ref[...]
        out = jax.lax.dot_general(h.astype(jnp.bfloat16), w2s[...], dn,
                                  preferred_element_type=jnp.float32)

        @pl.when(nh == 0)
        def _set():
            ys_ref[...] = out

        @pl.when(nh != 0)
        def _acc():
            ys_ref[...] += out


def _ffn_call(te, ntot, xs, wsort, W1b, W2b, W3b, mp, nt, interpret=False):
    grid_spec = pltpu.PrefetchScalarGridSpec(
        num_scalar_prefetch=2,
        grid=(NH, nt),
        in_specs=[
            pl.BlockSpec((TM, DIM), lambda nh, mt, te, ntot: (mt, 0)),
            pl.BlockSpec((TM, 1), lambda nh, mt, te, ntot: (mt, 0)),
            pl.BlockSpec((1, TH, DIM), lambda nh, mt, te, ntot:
                         (te[mt], nh, 0)),
            pl.BlockSpec((1, TH, DIM), lambda nh, mt, te, ntot:
                         (te[mt], nh, 0)),
            pl.BlockSpec((1, DIM, TH), lambda nh, mt, te, ntot:
                         (te[mt], 0, nh)),
        ],
        out_specs=pl.BlockSpec((TM, DIM), lambda nh, mt, te, ntot: (mt, 0)),
        scratch_shapes=[
            pltpu.VMEM((DIM, TH), jnp.bfloat16),
            pltpu.VMEM((DIM, TH), jnp.bfloat16),
            pltpu.VMEM((TH, DIM), jnp.bfloat16),
        ],
    )
    return pl.pallas_call(
        _ffn_kernel,
        grid_spec=grid_spec,
        out_shape=jax.ShapeDtypeStruct((mp, DIM), jnp.float32),
        compiler_params=pltpu.CompilerParams(
            dimension_semantics=("arbitrary", "arbitrary")),
        interpret=interpret,
    )(te, ntot, xs, wsort, W1b, W3b, W2b)


def _sc_dispatch(xt, pos2, mp, n):
    tpw = n // NW
    mesh = plsc.VectorSubcoreMesh(core_axis_name="c", subcore_axis_name="s")

    @functools.partial(
        pl.kernel, mesh=mesh,
        out_type=jax.ShapeDtypeStruct((mp, DIM), jnp.float32),
        scratch_types=[
            pltpu.VMEM((tpw,), jnp.int32),
            pltpu.VMEM((tpw,), jnp.int32),
            pltpu.VMEM((tpw, DIM), jnp.float32),
            pltpu.SemaphoreType.DMA,
        ],
    )
    def dispatch(x_hbm, pos_hbm, xs_hbm, idx0_v, idx1_v, rows_v, sem):
        wid = lax.axis_index("s") * NC + lax.axis_index("c")
        base = wid * tpw
        pltpu.sync_copy(pos_hbm.at[0, pl.ds(base, tpw)], idx0_v)
        pltpu.sync_copy(pos_hbm.at[1, pl.ds(base, tpw)], idx1_v)
        pltpu.sync_copy(x_hbm.at[pl.ds(base, tpw)], rows_v)
        pltpu.async_copy(rows_v, xs_hbm.at[idx0_v], sem).wait()
        pltpu.async_copy(rows_v, xs_hbm.at[idx1_v], sem).wait()

    return dispatch(xt, pos2)


def _sc_combine(ys, pos2, n):
    tpw = n // NW
    cch = tpw // 2
    mesh = plsc.VectorSubcoreMesh(core_axis_name="c", subcore_axis_name="s")

    @functools.partial(
        pl.kernel, mesh=mesh,
        out_type=jax.ShapeDtypeStruct((n, DIM), jnp.float32),
        scratch_types=[
            pltpu.VMEM((cch,), jnp.int32),
            pltpu.VMEM((cch,), jnp.int32),
            pltpu.VMEM((cch, DIM), jnp.float32),
            pltpu.VMEM((cch, DIM), jnp.float32),
            pltpu.SemaphoreType.DMA,
        ],
    )
    def combine(ys_hbm, pos_hbm, y_hbm, idx0_v, idx1_v, buf0, buf1, sem):
        wid = lax.axis_index("s") * NC + lax.axis_index("c")
        for c in range(2):
            base = wid * tpw + c * cch
            pltpu.sync_copy(pos_hbm.at[0, pl.ds(base, cch)], idx0_v)
            pltpu.sync_copy(pos_hbm.at[1, pl.ds(base, cch)], idx1_v)
            pltpu.async_copy(ys_hbm.at[idx0_v], buf0, sem).wait()
            pltpu.async_copy(ys_hbm.at[idx1_v], buf1, sem).wait()

            def row(i, _):
                def col(j, _):
                    sl = pl.ds(j * 16, 16)
                    buf0[i, sl] = buf0[i, sl] + buf1[i, sl]
                    return 0
                lax.fori_loop(0, DIM // 16, col, 0, unroll=8)
                return 0
            lax.fori_loop(0, cch, row, 0)
            pltpu.sync_copy(buf0, y_hbm.at[pl.ds(base, cch)])

    return combine(ys, pos2)


def _route(xt, gate_w, n):
    logits = xt @ gate_w.T
    scores = jax.nn.softmax(logits, axis=-1)
    topw, topi = jax.lax.top_k(scores, K)
    topw = topw / (jnp.sum(topw, axis=-1, keepdims=True) + 1e-20)
    e_flat = topi.T.reshape(-1)
    w_flat = topw.T.reshape(-1)
    onehot = (e_flat[:, None] == jnp.arange(E)[None, :]).astype(jnp.int32)
    ranks = jnp.cumsum(onehot, axis=0) - onehot
    rank = jnp.sum(ranks * onehot, axis=-1)
    counts = jnp.sum(onehot, axis=0)
    ntiles = (counts + TM - 1) // TM
    tile_start = jnp.concatenate(
        [jnp.zeros((1,), jnp.int32), jnp.cumsum(ntiles)[:-1]])
    pos = tile_start[e_flat] * TM + rank
    ntot = jnp.sum(ntiles)
    nt = n * K // TM + (E - 1)
    j = jnp.arange(nt)
    in_e = (j[:, None] >= tile_start[None, :]) & (
        j[:, None] < (tile_start + ntiles)[None, :])
    te_raw = jnp.sum(jnp.arange(E)[None, :] * in_e, axis=-1).astype(jnp.int32)
    te_last = jnp.sum(jnp.where(j == ntot - 1, te_raw, 0))
    te = jnp.where(j < ntot, te_raw, te_last).astype(jnp.int32)
    return pos, w_flat, te, ntot.astype(jnp.int32).reshape(1)


def kernel(x, W1, W2, W3, gate_w):
    orig_shape = x.shape
    xt = x.reshape(-1, DIM)
    n = xt.shape[0]
    nt = n * K // TM + (E - 1)
    mp = nt * TM

    pos, w_flat, te, ntot = _route(xt, gate_w, n)
    pos2 = pos.reshape(K, n)

    xs = _sc_dispatch(xt, pos2, mp, n)
    wsort = jnp.zeros((mp, 1), jnp.float32).at[pos, 0].set(w_flat)

    ys = _ffn_call(te, ntot, xs, wsort, W1, W2, W3, mp, nt)

    y = _sc_combine(ys, pos2, n)
    return y.reshape(orig_shape)
